# baseline (speedup 1.0000x reference)
import functools

import jax
import jax.numpy as jnp
from jax import lax
from jax.experimental import pallas as pl
from jax.experimental.pallas import tpu as pltpu
from jax.experimental.pallas import tpu_sc as plsc

_B, _N, _C, _K = 4, 2048, 64, 32
_R2 = 0.2 * 0.2
_L = 16
_NTILES = 32
_EPS = 1e-5


def _lba1_body(x_ref, w_ref, g_ref, b_ref, o_ref):
    xm = x_ref[...].reshape(_B * _N, _C)
    y = lax.dot_general(xm, w_ref[...], (((1,), (1,)), ((), ())),
                        preferred_element_type=jnp.float32)
    mean = jnp.mean(y, axis=0, keepdims=True)
    var = jnp.mean((y - mean) ** 2, axis=0, keepdims=True)
    x1 = jnp.maximum((y - mean) * lax.rsqrt(var + _EPS) * g_ref[...]
                     + b_ref[...], 0.0)
    o_ref[...] = x1.reshape(_B, _N, _C)


def _lba1(x, W1, g1, b1):
    return pl.pallas_call(
        _lba1_body,
        out_shape=jax.ShapeDtypeStruct((_B, _N, _C), jnp.float32),
    )(x, W1, g1[None, :], b1[None, :])


def _ball_query(pos_x, pos_y, pos_z):
    nq = (_B * _N) // _NTILES
    mesh = plsc.VectorSubcoreMesh(core_axis_name="c", subcore_axis_name="s")

    @functools.partial(
        pl.kernel,
        mesh=mesh,
        compiler_params=pltpu.CompilerParams(needs_layout_passes=False),
        out_type=jax.ShapeDtypeStruct((_B * _K * _N,), jnp.int32),
        scratch_types=[
            pltpu.VMEM((_N,), jnp.float32),
            pltpu.VMEM((_N,), jnp.float32),
            pltpu.VMEM((_N,), jnp.float32),
            pltpu.VMEM(((_K + 1) * nq,), jnp.int32),
            pltpu.SemaphoreType.DMA,
        ],
    )
    def k(posx_hbm, posy_hbm, posz_hbm, idx_hbm, px, py, pz, obuf, osem):
        wid = lax.axis_index("c") * 16 + lax.axis_index("s")
        q0 = wid * nq
        b = q0 // _N
        n0 = q0 % _N
        pltpu.sync_copy(posx_hbm.at[pl.ds(b * _N, _N)], px)
        pltpu.sync_copy(posy_hbm.at[pl.ds(b * _N, _N)], py)
        pltpu.sync_copy(posz_hbm.at[pl.ds(b * _N, _N)], pz)
        lanes = lax.iota(jnp.int32, _L)
        zeros = jnp.zeros((_L,), jnp.int32)

        strip = 256
        n_strips = _N // strip

        def per_group(g, _):
            qbase = n0 + g * _L
            qx = px[pl.ds(qbase, _L)]
            qy = py[pl.ds(qbase, _L)]
            qz = pz[pl.ds(qbase, _L)]
            qv = lanes + g * _L

            def cand(j, cnt_v):
                jv = jnp.full((_L,), j, jnp.int32)
                dx = qx - plsc.load_gather(px, [jv])
                dy = qy - plsc.load_gather(py, [jv])
                dz = qz - plsc.load_gather(pz, [jv])
                d2 = dx * dx + dy * dy + dz * dz
                m = d2 < _R2
                lin = jnp.minimum(cnt_v, _K) * nq + qv
                plsc.store_scatter(obuf, [lin], jv, mask=m)
                return cnt_v + jnp.where(m, 1, 0).astype(jnp.int32)

            def strip_cond(state):
                si, cnt_v = state
                return (si < n_strips) & (jnp.min(cnt_v) < _K)

            def strip_body(state):
                si, cnt_v = state
                base = si * strip

                new_cnt = plsc.parallel_loop(
                    0, strip, unroll=16, carry=cnt_v)(
                        lambda i, cv: cand(base + i, cv))
                return si + 1, new_cnt

            _si, cnt_v = lax.while_loop(strip_cond, strip_body,
                                        (jnp.int32(0), zeros))
            firsts = plsc.load_gather(obuf, [qv])
            for s in range(1, _K):
                plsc.store_scatter(obuf, [s * nq + qv], firsts,
                                   mask=(cnt_v <= s))
            return 0

        lax.fori_loop(0, nq // _L, per_group, 0)
        hs = [pltpu.async_copy(
                  obuf.at[pl.ds(s * nq, nq)],
                  idx_hbm.at[pl.ds((b * _K + s) * _N + n0, nq)], osem)
              for s in range(_K)]
        for h in hs:
            h.wait()

    return k(pos_x, pos_y, pos_z)


def _gather_max(x1t_flat, idx_t, pe):
    chunk_n = 256
    bn = _B * _N
    mesh = plsc.VectorSubcoreMesh(core_axis_name="c", subcore_axis_name="s")

    n_chunks_total = _B * (_N // chunk_n)

    @functools.partial(
        pl.kernel,
        mesh=mesh,
        compiler_params=pltpu.CompilerParams(needs_layout_passes=False),
        out_type=jax.ShapeDtypeStruct((_C * _B * _N,), jnp.float32),
        scratch_types=[
            pltpu.VMEM((_B * _N,), jnp.float32),
            pltpu.VMEM((_B * _N,), jnp.float32),
            pltpu.VMEM((2 * _K, chunk_n), jnp.int32),
            pltpu.VMEM((2, 2 * _K, chunk_n), jnp.float32),
            pltpu.VMEM((_B * _N,), jnp.float32),
            pltpu.VMEM((_B * _N,), jnp.float32),
            pltpu.SemaphoreType.DMA,
            pltpu.SemaphoreType.DMA,
        ],
    )
    def k(x_hbm, idx_hbm, pe_hbm, out_hbm,
          xrow0, xrow1, ichunk, pchunk, acc0r, acc1r, sem0, sem1):
        wid = lax.axis_index("c") * 16 + lax.axis_index("s")
        c0 = wid * 2
        sems = (sem0, sem1)

        def start(t):
            b, ci = divmod(t, _N // chunk_n)
            n0 = ci * chunk_n
            par = t % 2
            h1 = pltpu.async_copy(
                idx_hbm.at[b, :, pl.ds(n0, chunk_n)],
                ichunk.at[pl.ds(par * _K, _K), :], sems[par])
            h2 = pltpu.async_copy(
                pe_hbm.at[b, pl.ds(c0, 2), :, pl.ds(n0, chunk_n)],
                pchunk.at[:, pl.ds(par * _K, _K), :], sems[par])
            return (h1, h2)

        pltpu.sync_copy(x_hbm.at[pl.ds(c0 * bn, bn)], xrow0)
        pltpu.sync_copy(x_hbm.at[pl.ds((c0 + 1) * bn, bn)], xrow1)
        neg = jnp.full((_L,), -jnp.inf, jnp.float32)

        hs = start(0)
        for t in range(n_chunks_total):
            b, ci = divmod(t, _N // chunk_n)
            n0 = ci * chunk_n
            par = t % 2
            for h in hs:
                h.wait()
            if t + 1 < n_chunks_total:
                hs = start(t + 1)

            @plsc.parallel_loop(0, chunk_n // _L, unroll=2)
            def _groups(gi):
                base = gi * _L

                def do_slot(s, carry):
                    a0, a1 = carry
                    iv = ichunk[par * _K + s, pl.ds(base, _L)] + (b * _N)
                    g0 = plsc.load_gather(xrow0, [iv])
                    g1 = plsc.load_gather(xrow1, [iv])
                    p0 = pchunk[0, par * _K + s, pl.ds(base, _L)]
                    p1 = pchunk[1, par * _K + s, pl.ds(base, _L)]
                    return (jnp.maximum(a0, g0 + p0),
                            jnp.maximum(a1, g1 + p1))

                a0, a1 = lax.fori_loop(0, _K, do_slot, (neg, neg),
                                       unroll=8)
                col = b * _N + n0 + base
                acc0r[pl.ds(col, _L)] = a0
                acc1r[pl.ds(col, _L)] = a1

        pltpu.sync_copy(acc0r, out_hbm.at[pl.ds(c0 * bn, bn)])
        pltpu.sync_copy(acc1r, out_hbm.at[pl.ds((c0 + 1) * bn, bn)])

    return k(x1t_flat, idx_t, pe)


def _lba2_body(m_ref, x1t_ref, wa_ref, ga_ref, ba_ref,
               wb_ref, gb_ref, bb_ref, o_ref):
    m = m_ref[...]
    y = lax.dot_general(wa_ref[...], m, (((1,), (0,)), ((), ())),
                        preferred_element_type=jnp.float32)
    mean = jnp.mean(y, axis=1, keepdims=True)
    var = jnp.mean((y - mean) ** 2, axis=1, keepdims=True)
    h = jnp.maximum((y - mean) * lax.rsqrt(var + _EPS) * ga_ref[...]
                    + ba_ref[...], 0.0)
    y2 = lax.dot_general(wb_ref[...], h, (((1,), (0,)), ((), ())),
                         preferred_element_type=jnp.float32)
    mean2 = jnp.mean(y2, axis=1, keepdims=True)
    var2 = jnp.mean((y2 - mean2) ** 2, axis=1, keepdims=True)
    h2 = (y2 - mean2) * lax.rsqrt(var2 + _EPS) * gb_ref[...] + bb_ref[...]
    o_ref[...] = jnp.maximum(x1t_ref[...] + h2, 0.0)


def _lba2(maxed, x1t, W2a, g2a, b2a, W2b, g2b, b2b):
    return pl.pallas_call(
        _lba2_body,
        out_shape=jax.ShapeDtypeStruct((_C, _B * _N), jnp.float32),
    )(maxed, x1t, W2a, g2a[:, None], b2a[:, None],
      W2b, g2b[:, None], b2b[:, None])


def kernel(pos, x, pe, W1, g1, b1, W2a, g2a, b2a, W2b, g2b, b2b):
    x1 = _lba1(x, W1, g1, b1)
    x1t = jnp.transpose(x1, (2, 0, 1)).reshape(_C, _B * _N)
    pos_x = pos[:, :, 0].reshape(-1)
    pos_y = pos[:, :, 1].reshape(-1)
    pos_z = pos[:, :, 2].reshape(-1)
    idx_t = _ball_query(pos_x, pos_y, pos_z).reshape(_B, _K, _N)
    maxed = _gather_max(x1t.reshape(-1), idx_t, pe)
    out_t = _lba2(maxed.reshape(_C, _B * _N), x1t,
                  W2a, g2a, b2a, W2b, g2b, b2b)
    out_x = jnp.transpose(out_t.reshape(_C, _B, _N), (1, 2, 0))
    return (pos, out_x, pe)


# speedup vs baseline: 1.4227x; 1.4227x over previous
import functools

import jax
import jax.numpy as jnp
from jax import lax
from jax.experimental import pallas as pl
from jax.experimental.pallas import tpu as pltpu
from jax.experimental.pallas import tpu_sc as plsc

_B, _N, _C, _K = 4, 2048, 64, 32
_R2 = 0.2 * 0.2
_L = 16
_NTILES = 32
_EPS = 1e-5


def _lba1_body(x_ref, w_ref, g_ref, b_ref, o_ref):
    xm = x_ref[...].reshape(_B * _N, _C)
    y = lax.dot_general(xm, w_ref[...], (((1,), (1,)), ((), ())),
                        preferred_element_type=jnp.float32)
    mean = jnp.mean(y, axis=0, keepdims=True)
    var = jnp.mean((y - mean) ** 2, axis=0, keepdims=True)
    x1 = jnp.maximum((y - mean) * lax.rsqrt(var + _EPS) * g_ref[...]
                     + b_ref[...], 0.0)
    o_ref[...] = x1.T


def _lba1(x, W1, g1, b1):
    return pl.pallas_call(
        _lba1_body,
        out_shape=jax.ShapeDtypeStruct((_C, _B * _N), jnp.float32),
    )(x, W1, g1[None, :], b1[None, :])


def _ball_query(pos_x, pos_y, pos_z):
    nq = (_B * _N) // _NTILES
    mesh = plsc.VectorSubcoreMesh(core_axis_name="c", subcore_axis_name="s")

    @functools.partial(
        pl.kernel,
        mesh=mesh,
        compiler_params=pltpu.CompilerParams(needs_layout_passes=False),
        out_type=jax.ShapeDtypeStruct((_B * _K * _N,), jnp.int32),
        scratch_types=[
            pltpu.VMEM((_N,), jnp.float32),
            pltpu.VMEM((_N,), jnp.float32),
            pltpu.VMEM((_N,), jnp.float32),
            pltpu.VMEM(((_K + 1) * nq,), jnp.int32),
            pltpu.SemaphoreType.DMA,
        ],
    )
    def k(posx_hbm, posy_hbm, posz_hbm, idx_hbm, px, py, pz, obuf, osem):
        wid = lax.axis_index("c") * 16 + lax.axis_index("s")
        q0 = wid * nq
        b = q0 // _N
        n0 = q0 % _N
        pltpu.sync_copy(posx_hbm.at[pl.ds(b * _N, _N)], px)
        pltpu.sync_copy(posy_hbm.at[pl.ds(b * _N, _N)], py)
        pltpu.sync_copy(posz_hbm.at[pl.ds(b * _N, _N)], pz)
        lanes = lax.iota(jnp.int32, _L)
        zeros = jnp.zeros((_L,), jnp.int32)

        strip = 128
        n_strips = _N // strip

        def per_group(g, _):
            qbase = n0 + g * _L
            qx = px[pl.ds(qbase, _L)]
            qy = py[pl.ds(qbase, _L)]
            qz = pz[pl.ds(qbase, _L)]
            qv = lanes + g * _L

            def cand(j, cnt_v):
                jv = jnp.full((_L,), j, jnp.int32)
                dx = qx - plsc.load_gather(px, [jv])
                dy = qy - plsc.load_gather(py, [jv])
                dz = qz - plsc.load_gather(pz, [jv])
                d2 = dx * dx + dy * dy + dz * dz
                m = d2 < _R2
                lin = jnp.minimum(cnt_v, _K) * nq + qv
                plsc.store_scatter(obuf, [lin], jv, mask=m)
                return cnt_v + jnp.where(m, 1, 0).astype(jnp.int32)

            def strip_cond(state):
                si, cnt_v = state
                return (si < n_strips) & (jnp.min(cnt_v) < _K)

            def strip_body(state):
                si, cnt_v = state
                base = si * strip

                new_cnt = plsc.parallel_loop(
                    0, strip, unroll=8, carry=cnt_v)(
                        lambda i, cv: cand(base + i, cv))
                return si + 1, new_cnt

            _si, cnt_v = lax.while_loop(strip_cond, strip_body,
                                        (jnp.int32(0), zeros))
            firsts = plsc.load_gather(obuf, [qv])
            for s in range(1, _K):
                plsc.store_scatter(obuf, [s * nq + qv], firsts,
                                   mask=(cnt_v <= s))
            return 0

        lax.fori_loop(0, nq // _L, per_group, 0)
        hs = [pltpu.async_copy(
                  obuf.at[pl.ds(s * nq, nq)],
                  idx_hbm.at[pl.ds((b * _K + s) * _N + n0, nq)], osem)
              for s in range(_K)]
        for h in hs:
            h.wait()

    return k(pos_x, pos_y, pos_z)


def _gather_max(x1t_flat, idx_t, pe):
    chunk_n = 256
    bn = _B * _N
    mesh = plsc.VectorSubcoreMesh(core_axis_name="c", subcore_axis_name="s")

    n_chunks_total = _B * (_N // chunk_n)

    @functools.partial(
        pl.kernel,
        mesh=mesh,
        compiler_params=pltpu.CompilerParams(needs_layout_passes=False),
        out_type=jax.ShapeDtypeStruct((_C * _B * _N,), jnp.float32),
        scratch_types=[
            pltpu.VMEM((_B * _N,), jnp.float32),
            pltpu.VMEM((_B * _N,), jnp.float32),
            pltpu.VMEM((2 * _K, chunk_n), jnp.int32),
            pltpu.VMEM((2, 2 * _K, chunk_n), jnp.float32),
            pltpu.VMEM((_B * _N,), jnp.float32),
            pltpu.VMEM((_B * _N,), jnp.float32),
            pltpu.SemaphoreType.DMA,
            pltpu.SemaphoreType.DMA,
        ],
    )
    def k(x_hbm, idx_hbm, pe_hbm, out_hbm,
          xrow0, xrow1, ichunk, pchunk, acc0r, acc1r, sem0, sem1):
        wid = lax.axis_index("c") * 16 + lax.axis_index("s")
        c0 = wid * 2
        sems = (sem0, sem1)

        def start(t):
            b, ci = divmod(t, _N // chunk_n)
            n0 = ci * chunk_n
            par = t % 2
            h1 = pltpu.async_copy(
                idx_hbm.at[b, :, pl.ds(n0, chunk_n)],
                ichunk.at[pl.ds(par * _K, _K), :], sems[par])
            h2 = pltpu.async_copy(
                pe_hbm.at[b, pl.ds(c0, 2), :, pl.ds(n0, chunk_n)],
                pchunk.at[:, pl.ds(par * _K, _K), :], sems[par])
            return (h1, h2)

        pltpu.sync_copy(x_hbm.at[pl.ds(c0 * bn, bn)], xrow0)
        pltpu.sync_copy(x_hbm.at[pl.ds((c0 + 1) * bn, bn)], xrow1)
        neg = jnp.full((_L,), -jnp.inf, jnp.float32)

        hs = start(0)
        for t in range(n_chunks_total):
            b, ci = divmod(t, _N // chunk_n)
            n0 = ci * chunk_n
            par = t % 2
            for h in hs:
                h.wait()
            if t + 1 < n_chunks_total:
                hs = start(t + 1)

            @plsc.parallel_loop(0, chunk_n // _L, unroll=2)
            def _groups(gi):
                base = gi * _L

                def do_slot(s, carry):
                    a0, a1 = carry
                    iv = ichunk[par * _K + s, pl.ds(base, _L)] + (b * _N)
                    g0 = plsc.load_gather(xrow0, [iv])
                    g1 = plsc.load_gather(xrow1, [iv])
                    p0 = pchunk[0, par * _K + s, pl.ds(base, _L)]
                    p1 = pchunk[1, par * _K + s, pl.ds(base, _L)]
                    return (jnp.maximum(a0, g0 + p0),
                            jnp.maximum(a1, g1 + p1))

                a0, a1 = lax.fori_loop(0, _K, do_slot, (neg, neg),
                                       unroll=8)
                col = b * _N + n0 + base
                acc0r[pl.ds(col, _L)] = a0
                acc1r[pl.ds(col, _L)] = a1

        pltpu.sync_copy(acc0r, out_hbm.at[pl.ds(c0 * bn, bn)])
        pltpu.sync_copy(acc1r, out_hbm.at[pl.ds((c0 + 1) * bn, bn)])

    return k(x1t_flat, idx_t, pe)


def _lba2_body(m_ref, x1t_ref, wa_ref, ga_ref, ba_ref,
               wb_ref, gb_ref, bb_ref, o_ref):
    m = m_ref[...]
    y = lax.dot_general(wa_ref[...], m, (((1,), (0,)), ((), ())),
                        preferred_element_type=jnp.float32)
    mean = jnp.mean(y, axis=1, keepdims=True)
    var = jnp.mean((y - mean) ** 2, axis=1, keepdims=True)
    h = jnp.maximum((y - mean) * lax.rsqrt(var + _EPS) * ga_ref[...]
                    + ba_ref[...], 0.0)
    y2 = lax.dot_general(wb_ref[...], h, (((1,), (0,)), ((), ())),
                         preferred_element_type=jnp.float32)
    mean2 = jnp.mean(y2, axis=1, keepdims=True)
    var2 = jnp.mean((y2 - mean2) ** 2, axis=1, keepdims=True)
    h2 = (y2 - mean2) * lax.rsqrt(var2 + _EPS) * gb_ref[...] + bb_ref[...]
    out_t = jnp.maximum(x1t_ref[...] + h2, 0.0)
    o_ref[...] = out_t.T.reshape(_B, _N, _C)


def _lba2(maxed, x1t, W2a, g2a, b2a, W2b, g2b, b2b):
    return pl.pallas_call(
        _lba2_body,
        out_shape=jax.ShapeDtypeStruct((_B, _N, _C), jnp.float32),
    )(maxed, x1t, W2a, g2a[:, None], b2a[:, None],
      W2b, g2b[:, None], b2b[:, None])


def kernel(pos, x, pe, W1, g1, b1, W2a, g2a, b2a, W2b, g2b, b2b):
    x1t = _lba1(x, W1, g1, b1)
    pos_x = pos[:, :, 0].reshape(-1)
    pos_y = pos[:, :, 1].reshape(-1)
    pos_z = pos[:, :, 2].reshape(-1)
    idx_t = _ball_query(pos_x, pos_y, pos_z).reshape(_B, _K, _N)
    maxed = _gather_max(x1t.reshape(-1), idx_t, pe)
    out_x = _lba2(maxed.reshape(_C, _B * _N), x1t,
                  W2a, g2a, b2a, W2b, g2b, b2b)
    return (pos, out_x, pe)


# speedup vs baseline: 1.4361x; 1.0094x over previous
import functools

import jax
import jax.numpy as jnp
from jax import lax
from jax.experimental import pallas as pl
from jax.experimental.pallas import tpu as pltpu
from jax.experimental.pallas import tpu_sc as plsc

_B, _N, _C, _K = 4, 2048, 64, 32
_R2 = 0.2 * 0.2
_L = 16
_NTILES = 32
_EPS = 1e-5


def _lba1_body(x_ref, w_ref, g_ref, b_ref, o_ref):
    xm = x_ref[...].reshape(_B * _N, _C)
    y = lax.dot_general(xm, w_ref[...], (((1,), (1,)), ((), ())),
                        preferred_element_type=jnp.float32)
    mean = jnp.mean(y, axis=0, keepdims=True)
    var = jnp.mean((y - mean) ** 2, axis=0, keepdims=True)
    x1 = jnp.maximum((y - mean) * lax.rsqrt(var + _EPS) * g_ref[...]
                     + b_ref[...], 0.0)
    o_ref[...] = x1.reshape(_B, _N, _C)


def _lba1(x, W1, g1, b1):
    return pl.pallas_call(
        _lba1_body,
        out_shape=jax.ShapeDtypeStruct((_B, _N, _C), jnp.float32),
    )(x, W1, g1[None, :], b1[None, :])


def _ball_query(pos_x, pos_y, pos_z):
    nq = (_B * _N) // _NTILES
    mesh = plsc.VectorSubcoreMesh(core_axis_name="c", subcore_axis_name="s")

    @functools.partial(
        pl.kernel,
        mesh=mesh,
        compiler_params=pltpu.CompilerParams(needs_layout_passes=False),
        out_type=jax.ShapeDtypeStruct((_B * _K * _N,), jnp.int32),
        scratch_types=[
            pltpu.VMEM((_N,), jnp.float32),
            pltpu.VMEM((_N,), jnp.float32),
            pltpu.VMEM((_N,), jnp.float32),
            pltpu.VMEM(((_K + 1) * nq,), jnp.int32),
            pltpu.SemaphoreType.DMA,
        ],
    )
    def k(posx_hbm, posy_hbm, posz_hbm, idx_hbm, px, py, pz, obuf, osem):
        wid = lax.axis_index("c") * 16 + lax.axis_index("s")
        q0 = wid * nq
        b = q0 // _N
        n0 = q0 % _N
        pltpu.sync_copy(posx_hbm.at[pl.ds(b * _N, _N)], px)
        pltpu.sync_copy(posy_hbm.at[pl.ds(b * _N, _N)], py)
        pltpu.sync_copy(posz_hbm.at[pl.ds(b * _N, _N)], pz)
        lanes = lax.iota(jnp.int32, _L)
        zeros = jnp.zeros((_L,), jnp.int32)

        strip = 128
        n_strips = _N // strip

        def per_group(g, _):
            qbase = n0 + g * _L
            qx = px[pl.ds(qbase, _L)]
            qy = py[pl.ds(qbase, _L)]
            qz = pz[pl.ds(qbase, _L)]
            qv = lanes + g * _L

            def cand(j, cnt_v):
                jv = jnp.full((_L,), j, jnp.int32)
                dx = qx - plsc.load_gather(px, [jv])
                dy = qy - plsc.load_gather(py, [jv])
                dz = qz - plsc.load_gather(pz, [jv])
                d2 = dx * dx + dy * dy + dz * dz
                m = d2 < _R2
                lin = jnp.minimum(cnt_v, _K) * nq + qv
                plsc.store_scatter(obuf, [lin], jv, mask=m)
                return cnt_v + jnp.where(m, 1, 0).astype(jnp.int32)

            def strip_cond(state):
                si, cnt_v = state
                return (si < n_strips) & (jnp.min(cnt_v) < _K)

            def strip_body(state):
                si, cnt_v = state
                base = si * strip

                new_cnt = plsc.parallel_loop(
                    0, strip, unroll=8, carry=cnt_v)(
                        lambda i, cv: cand(base + i, cv))
                return si + 1, new_cnt

            _si, cnt_v = lax.while_loop(strip_cond, strip_body,
                                        (jnp.int32(0), zeros))
            firsts = plsc.load_gather(obuf, [qv])
            for s in range(1, _K):
                plsc.store_scatter(obuf, [s * nq + qv], firsts,
                                   mask=(cnt_v <= s))
            return 0

        lax.fori_loop(0, nq // _L, per_group, 0)
        hs = [pltpu.async_copy(
                  obuf.at[pl.ds(s * nq, nq)],
                  idx_hbm.at[pl.ds((b * _K + s) * _N + n0, nq)], osem)
              for s in range(_K)]
        for h in hs:
            h.wait()

    return k(pos_x, pos_y, pos_z)


def _gather_max(x1t_flat, idx_t, pe):
    chunk_n = 256
    bn = _B * _N
    mesh = plsc.VectorSubcoreMesh(core_axis_name="c", subcore_axis_name="s")

    n_chunks_total = _B * (_N // chunk_n)

    @functools.partial(
        pl.kernel,
        mesh=mesh,
        compiler_params=pltpu.CompilerParams(needs_layout_passes=False),
        out_type=jax.ShapeDtypeStruct((_C * _B * _N,), jnp.float32),
        scratch_types=[
            pltpu.VMEM((_B * _N,), jnp.float32),
            pltpu.VMEM((_B * _N,), jnp.float32),
            pltpu.VMEM((2 * _K, chunk_n), jnp.int32),
            pltpu.VMEM((2, 2 * _K, chunk_n), jnp.float32),
            pltpu.VMEM((_B * _N,), jnp.float32),
            pltpu.VMEM((_B * _N,), jnp.float32),
            pltpu.SemaphoreType.DMA,
            pltpu.SemaphoreType.DMA,
        ],
    )
    def k(x_hbm, idx_hbm, pe_hbm, out_hbm,
          xrow0, xrow1, ichunk, pchunk, acc0r, acc1r, sem0, sem1):
        wid = lax.axis_index("c") * 16 + lax.axis_index("s")
        c0 = wid * 2
        sems = (sem0, sem1)

        def start(t):
            b, ci = divmod(t, _N // chunk_n)
            n0 = ci * chunk_n
            par = t % 2
            h1 = pltpu.async_copy(
                idx_hbm.at[b, :, pl.ds(n0, chunk_n)],
                ichunk.at[pl.ds(par * _K, _K), :], sems[par])
            h2 = pltpu.async_copy(
                pe_hbm.at[b, pl.ds(c0, 2), :, pl.ds(n0, chunk_n)],
                pchunk.at[:, pl.ds(par * _K, _K), :], sems[par])
            return (h1, h2)

        pltpu.sync_copy(x_hbm.at[pl.ds(c0 * bn, bn)], xrow0)
        pltpu.sync_copy(x_hbm.at[pl.ds((c0 + 1) * bn, bn)], xrow1)
        neg = jnp.full((_L,), -jnp.inf, jnp.float32)

        hs = start(0)
        for t in range(n_chunks_total):
            b, ci = divmod(t, _N // chunk_n)
            n0 = ci * chunk_n
            par = t % 2
            for h in hs:
                h.wait()
            if t + 1 < n_chunks_total:
                hs = start(t + 1)

            @plsc.parallel_loop(0, chunk_n // _L, unroll=2)
            def _groups(gi):
                base = gi * _L

                def do_slot(s, carry):
                    a0, a1 = carry
                    iv = ichunk[par * _K + s, pl.ds(base, _L)] + (b * _N)
                    g0 = plsc.load_gather(xrow0, [iv])
                    g1 = plsc.load_gather(xrow1, [iv])
                    p0 = pchunk[0, par * _K + s, pl.ds(base, _L)]
                    p1 = pchunk[1, par * _K + s, pl.ds(base, _L)]
                    return (jnp.maximum(a0, g0 + p0),
                            jnp.maximum(a1, g1 + p1))

                a0, a1 = lax.fori_loop(0, _K, do_slot, (neg, neg),
                                       unroll=8)
                col = b * _N + n0 + base
                acc0r[pl.ds(col, _L)] = a0
                acc1r[pl.ds(col, _L)] = a1

        pltpu.sync_copy(acc0r, out_hbm.at[pl.ds(c0 * bn, bn)])
        pltpu.sync_copy(acc1r, out_hbm.at[pl.ds((c0 + 1) * bn, bn)])

    return k(x1t_flat, idx_t, pe)


def _lba2_body(m_ref, x1t_ref, wa_ref, ga_ref, ba_ref,
               wb_ref, gb_ref, bb_ref, o_ref):
    m = m_ref[...]
    y = lax.dot_general(wa_ref[...], m, (((1,), (0,)), ((), ())),
                        preferred_element_type=jnp.float32)
    mean = jnp.mean(y, axis=1, keepdims=True)
    var = jnp.mean((y - mean) ** 2, axis=1, keepdims=True)
    h = jnp.maximum((y - mean) * lax.rsqrt(var + _EPS) * ga_ref[...]
                    + ba_ref[...], 0.0)
    y2 = lax.dot_general(wb_ref[...], h, (((1,), (0,)), ((), ())),
                         preferred_element_type=jnp.float32)
    mean2 = jnp.mean(y2, axis=1, keepdims=True)
    var2 = jnp.mean((y2 - mean2) ** 2, axis=1, keepdims=True)
    h2 = (y2 - mean2) * lax.rsqrt(var2 + _EPS) * gb_ref[...] + bb_ref[...]
    o_ref[...] = jnp.maximum(x1t_ref[...] + h2, 0.0)


def _lba2(maxed, x1t, W2a, g2a, b2a, W2b, g2b, b2b):
    return pl.pallas_call(
        _lba2_body,
        out_shape=jax.ShapeDtypeStruct((_C, _B * _N), jnp.float32),
    )(maxed, x1t, W2a, g2a[:, None], b2a[:, None],
      W2b, g2b[:, None], b2b[:, None])


def kernel(pos, x, pe, W1, g1, b1, W2a, g2a, b2a, W2b, g2b, b2b):
    x1 = _lba1(x, W1, g1, b1)
    x1t = jnp.transpose(x1, (2, 0, 1)).reshape(_C, _B * _N)
    pos_x = pos[:, :, 0].reshape(-1)
    pos_y = pos[:, :, 1].reshape(-1)
    pos_z = pos[:, :, 2].reshape(-1)
    idx_t = _ball_query(pos_x, pos_y, pos_z).reshape(_B, _K, _N)
    maxed = _gather_max(x1t.reshape(-1), idx_t, pe)
    out_t = _lba2(maxed.reshape(_C, _B * _N), x1t,
                  W2a, g2a, b2a, W2b, g2b, b2b)
    out_x = jnp.transpose(out_t.reshape(_C, _B, _N), (1, 2, 0))
    return (pos, out_x, pe)
